# baseline (speedup 1.0000x reference)
import functools

import jax
import jax.numpy as jnp
from jax import lax
from jax.experimental import pallas as pl
from jax.experimental.pallas import tpu as pltpu
from jax.experimental.pallas import tpu_sc as plsc

K = 8
BLOCK_V = 512

_NEG_PAD = -2.0
_NEG_DONE = -3.0
_BIG_I32 = 2**30


def _l2n(x):
    n = jnp.sqrt(jnp.sum(x * x, axis=1, keepdims=True))
    return x / jnp.maximum(n, 1e-12)


def _qnorm_kernel(q_ref, qn_ref):
    qn_ref[...] = _l2n(q_ref[...]).astype(jnp.bfloat16)


def _simblock_kernel(vocab_size, qn_ref, v_ref, sims_ref, bm_ref):
    j = pl.program_id(0)
    vn = _l2n(v_ref[...]).astype(jnp.bfloat16)
    sims = jax.lax.dot_general(
        qn_ref[...], vn, dimension_numbers=(((1,), (1,)), ((), ())),
        preferred_element_type=jnp.float32)
    col = j * BLOCK_V + jax.lax.broadcasted_iota(jnp.int32, sims.shape, 1)
    sims = jnp.where(col >= vocab_size, _NEG_PAD, sims)
    sims_ref[0] = sims
    bm_ref[0] = jnp.max(sims, axis=1, keepdims=True)


def _blocksel_kernel(bm_ref, sel_ref, row_ref):
    bm = bm_ref[...]
    n = bm.shape[0]
    bid = jax.lax.broadcasted_iota(jnp.int32, bm.shape, 1)
    sels = []
    for _ in range(K):
        m = jnp.max(bm, axis=1)
        ci = jnp.min(jnp.where(bm == m[:, None], bid, _BIG_I32), axis=1)
        sels.append(ci)
        bm = jnp.where(bid == ci[:, None], _NEG_DONE, bm)
    sel = jnp.stack(sels, axis=0)
    sel_ref[...] = sel
    qi = jax.lax.broadcasted_iota(jnp.int32, (K, n), 1)
    row_ref[...] = sel * n + qi


def _final_kernel(g_ref, sel_ref, vals_ref, idx_ref):
    nh = sel_ref.shape[0]
    g = g_ref[...].reshape(nh, K, BLOCK_V)
    sel = sel_ref[...]
    lane = jax.lax.broadcasted_iota(jnp.int32, (nh, K, BLOCK_V), 2)
    cols = sel[:, :, None] * BLOCK_V + lane
    vals, idxs = [], []
    for _ in range(K):
        m = jnp.max(jnp.max(g, axis=2), axis=1)
        cand = jnp.where(g == m[:, None, None], cols, _BIG_I32)
        ci = jnp.min(jnp.min(cand, axis=2), axis=1)
        vals.append(m)
        idxs.append(ci)
        g = jnp.where(cols == ci[:, None, None], _NEG_DONE, g)
    vals_ref[0] = jnp.stack(vals, axis=0)
    idx_ref[0] = jnp.stack(idxs, axis=0)


def _sc_info():
    try:
        info = plsc.get_sparse_core_info()
        return info.num_cores, info.num_subcores
    except Exception:
        return 2, 16


def _sc_gather(table, rowids, n):
    nc, ns = _sc_info()
    nw = nc * ns
    nk = n * K
    b_per_w = nk // nw
    gsub = min(b_per_w, 16)
    nsub = b_per_w // gsub
    mesh = plsc.VectorSubcoreMesh(core_axis_name="c", subcore_axis_name="s")

    @functools.partial(
        pl.kernel, mesh=mesh,
        out_type=jax.ShapeDtypeStruct((nk, BLOCK_V), jnp.float32),
        scratch_types=[
            pltpu.VMEM((b_per_w,), jnp.int32),
            pltpu.VMEM((gsub, BLOCK_V), jnp.float32),
            pltpu.VMEM((gsub, BLOCK_V), jnp.float32),
            pltpu.SemaphoreType.DMA,
            pltpu.SemaphoreType.DMA,
        ],
    )
    def gather_kernel(table_hbm, idx_hbm, out_hbm, idx_v, rows_a, rows_b,
                      sem_a, sem_b):
        wid = lax.axis_index("s") * nc + lax.axis_index("c")
        base = wid * b_per_w
        pltpu.sync_copy(idx_hbm.at[pl.ds(base, b_per_w)], idx_v)
        bufs = (rows_a, rows_b)
        sems = (sem_a, sem_b)
        cps = []
        for i in range(nsub):
            cps.append(pltpu.async_copy(
                table_hbm.at[idx_v.at[pl.ds(i * gsub, gsub)]],
                bufs[i % 2], sems[i % 2]))
            if i > 0:
                cps[i - 1].wait()
                pltpu.sync_copy(bufs[(i - 1) % 2],
                                out_hbm.at[pl.ds(base + (i - 1) * gsub, gsub)])
        cps[nsub - 1].wait()
        pltpu.sync_copy(bufs[(nsub - 1) % 2],
                        out_hbm.at[pl.ds(base + (nsub - 1) * gsub, gsub)])

    return gather_kernel(table, rowids)


def kernel(embeddings, vocab_embeddings):
    orig_shape = embeddings.shape[:-1]
    d = embeddings.shape[-1]
    q = embeddings.reshape(-1, d)
    n = q.shape[0]
    vocab_size = vocab_embeddings.shape[0]
    nbh = (vocab_size + 2 * BLOCK_V - 1) // (2 * BLOCK_V)
    nb = 2 * nbh
    vpad = nb * BLOCK_V
    v = jnp.pad(vocab_embeddings, ((0, vpad - vocab_size), (0, 0)))

    qn = pl.pallas_call(
        _qnorm_kernel,
        out_shape=jax.ShapeDtypeStruct((n, d), jnp.bfloat16),
    )(q)

    sims, bm = pl.pallas_call(
        functools.partial(_simblock_kernel, vocab_size),
        grid=(nb,),
        in_specs=[
            pl.BlockSpec((n, d), lambda j: (0, 0)),
            pl.BlockSpec((BLOCK_V, d), lambda j: (j, 0)),
        ],
        out_specs=[
            pl.BlockSpec((1, n, BLOCK_V), lambda j: (j, 0, 0)),
            pl.BlockSpec((1, n, 1), lambda j: (j, 0, 0)),
        ],
        out_shape=[
            jax.ShapeDtypeStruct((nb, n, BLOCK_V), jnp.float32),
            jax.ShapeDtypeStruct((nb, n, 1), jnp.float32),
        ],
        compiler_params=pltpu.CompilerParams(
            dimension_semantics=("parallel",)),
    )(qn, v)
    bm_t = bm[:, :, 0].T

    sel, rows = pl.pallas_call(
        _blocksel_kernel,
        out_shape=[
            jax.ShapeDtypeStruct((K, n), jnp.int32),
            jax.ShapeDtypeStruct((K, n), jnp.int32),
        ],
    )(bm_t)
    sel_t = sel.T
    rowids = rows.T.reshape(n * K)

    table = sims.reshape(nb * n, BLOCK_V)
    gath = _sc_gather(table, rowids, n)

    nqt = min(n, 64)
    nt = n // (2 * nqt)
    fvals, fidx = pl.pallas_call(
        _final_kernel,
        grid=(2, nt),
        in_specs=[
            pl.BlockSpec((nqt * K, BLOCK_V), lambda h, t: (h * nt + t, 0)),
            pl.BlockSpec((nqt, K), lambda h, t: (h * nt + t, 0)),
        ],
        out_specs=[
            pl.BlockSpec((1, K, nqt), lambda h, t: (h * nt + t, 0, 0)),
            pl.BlockSpec((1, K, nqt), lambda h, t: (h * nt + t, 0, 0)),
        ],
        out_shape=[
            jax.ShapeDtypeStruct((2 * nt, K, nqt), jnp.float32),
            jax.ShapeDtypeStruct((2 * nt, K, nqt), jnp.int32),
        ],
        compiler_params=pltpu.CompilerParams(
            dimension_semantics=("parallel", "arbitrary")),
    )(gath, sel_t)

    fv = fvals.transpose(0, 2, 1).reshape(n, K)
    fi = fidx.transpose(0, 2, 1).reshape(n, K)
    topk_sim = fv.reshape(orig_shape + (K,))
    topk_idx = fi.reshape(orig_shape + (K,))
    token_ids = fi[:, 0].reshape(orig_shape)
    return (token_ids, topk_sim, topk_idx)


# speedup vs baseline: 1.1581x; 1.1581x over previous
import functools

import jax
import jax.numpy as jnp
from jax import lax
from jax.experimental import pallas as pl
from jax.experimental.pallas import tpu as pltpu
from jax.experimental.pallas import tpu_sc as plsc

K = 8
BLOCK_V = 1024

_NEG_PAD = -2.0
_NEG_DONE = -3.0
_BIG_I32 = 2**30


def _l2n(x):
    n = jnp.sqrt(jnp.sum(x * x, axis=1, keepdims=True))
    return x / jnp.maximum(n, 1e-12)


def _qnorm_kernel(q_ref, qn_ref):
    qn_ref[...] = _l2n(q_ref[...]).astype(jnp.bfloat16)


def _simblock_kernel(vocab_size, qn_ref, v_ref, sims_ref, bm_ref):
    j = pl.program_id(0)
    vn = _l2n(v_ref[...]).astype(jnp.bfloat16)
    sims = jax.lax.dot_general(
        qn_ref[...], vn, dimension_numbers=(((1,), (1,)), ((), ())),
        preferred_element_type=jnp.float32)
    col = j * BLOCK_V + jax.lax.broadcasted_iota(jnp.int32, sims.shape, 1)
    sims = jnp.where(col >= vocab_size, _NEG_PAD, sims)
    sims_ref[0] = sims
    bm_ref[0] = jnp.max(sims, axis=1, keepdims=True)


def _blocksel_kernel(bm_ref, sel_ref, row_ref):
    bm = bm_ref[...]
    n = bm.shape[0]
    bid = jax.lax.broadcasted_iota(jnp.int32, bm.shape, 1)
    sels = []
    for _ in range(K):
        m = jnp.max(bm, axis=1)
        ci = jnp.min(jnp.where(bm == m[:, None], bid, _BIG_I32), axis=1)
        sels.append(ci)
        bm = jnp.where(bid == ci[:, None], _NEG_DONE, bm)
    sel = jnp.stack(sels, axis=0)
    sel_ref[...] = sel
    qi = jax.lax.broadcasted_iota(jnp.int32, (K, n), 1)
    row_ref[...] = sel * n + qi


def _final_kernel(g_ref, sel_ref, vals_ref, idx_ref):
    nh = sel_ref.shape[0]
    g = g_ref[...].reshape(nh, K, BLOCK_V)
    sel = sel_ref[...]
    lane = jax.lax.broadcasted_iota(jnp.int32, (nh, K, BLOCK_V), 2)
    cols = sel[:, :, None] * BLOCK_V + lane
    vals, idxs = [], []
    for _ in range(K):
        m = jnp.max(jnp.max(g, axis=2), axis=1)
        cand = jnp.where(g == m[:, None, None], cols, _BIG_I32)
        ci = jnp.min(jnp.min(cand, axis=2), axis=1)
        vals.append(m)
        idxs.append(ci)
        g = jnp.where(cols == ci[:, None, None], _NEG_DONE, g)
    vals_ref[0] = jnp.stack(vals, axis=0)
    idx_ref[0] = jnp.stack(idxs, axis=0)


def _sc_info():
    try:
        info = plsc.get_sparse_core_info()
        return info.num_cores, info.num_subcores
    except Exception:
        return 2, 16


def _sc_gather(table, rowids, n):
    nc, ns = _sc_info()
    nw = nc * ns
    nk = n * K
    b_per_w = nk // nw
    gsub = min(b_per_w, 16)
    nsub = b_per_w // gsub
    mesh = plsc.VectorSubcoreMesh(core_axis_name="c", subcore_axis_name="s")

    @functools.partial(
        pl.kernel, mesh=mesh,
        out_type=jax.ShapeDtypeStruct((nk, BLOCK_V), jnp.float32),
        scratch_types=[
            pltpu.VMEM((b_per_w,), jnp.int32),
            pltpu.VMEM((gsub, BLOCK_V), jnp.float32),
            pltpu.VMEM((gsub, BLOCK_V), jnp.float32),
            pltpu.SemaphoreType.DMA,
            pltpu.SemaphoreType.DMA,
        ],
    )
    def gather_kernel(table_hbm, idx_hbm, out_hbm, idx_v, rows_a, rows_b,
                      sem_a, sem_b):
        wid = lax.axis_index("s") * nc + lax.axis_index("c")
        base = wid * b_per_w
        pltpu.sync_copy(idx_hbm.at[pl.ds(base, b_per_w)], idx_v)
        bufs = (rows_a, rows_b)
        sems = (sem_a, sem_b)
        cps = []
        for i in range(nsub):
            cps.append(pltpu.async_copy(
                table_hbm.at[idx_v.at[pl.ds(i * gsub, gsub)]],
                bufs[i % 2], sems[i % 2]))
            if i > 0:
                cps[i - 1].wait()
                pltpu.sync_copy(bufs[(i - 1) % 2],
                                out_hbm.at[pl.ds(base + (i - 1) * gsub, gsub)])
        cps[nsub - 1].wait()
        pltpu.sync_copy(bufs[(nsub - 1) % 2],
                        out_hbm.at[pl.ds(base + (nsub - 1) * gsub, gsub)])

    return gather_kernel(table, rowids)


def kernel(embeddings, vocab_embeddings):
    orig_shape = embeddings.shape[:-1]
    d = embeddings.shape[-1]
    q = embeddings.reshape(-1, d)
    n = q.shape[0]
    vocab_size = vocab_embeddings.shape[0]
    nbh = (vocab_size + 2 * BLOCK_V - 1) // (2 * BLOCK_V)
    nb = 2 * nbh
    vpad = nb * BLOCK_V
    v = jnp.pad(vocab_embeddings, ((0, vpad - vocab_size), (0, 0)))

    qn = pl.pallas_call(
        _qnorm_kernel,
        out_shape=jax.ShapeDtypeStruct((n, d), jnp.bfloat16),
    )(q)

    sims, bm = pl.pallas_call(
        functools.partial(_simblock_kernel, vocab_size),
        grid=(nb,),
        in_specs=[
            pl.BlockSpec((n, d), lambda j: (0, 0)),
            pl.BlockSpec((BLOCK_V, d), lambda j: (j, 0)),
        ],
        out_specs=[
            pl.BlockSpec((1, n, BLOCK_V), lambda j: (j, 0, 0)),
            pl.BlockSpec((1, n, 1), lambda j: (j, 0, 0)),
        ],
        out_shape=[
            jax.ShapeDtypeStruct((nb, n, BLOCK_V), jnp.float32),
            jax.ShapeDtypeStruct((nb, n, 1), jnp.float32),
        ],
        compiler_params=pltpu.CompilerParams(
            dimension_semantics=("parallel",)),
    )(qn, v)
    bm_t = bm[:, :, 0].T

    sel, rows = pl.pallas_call(
        _blocksel_kernel,
        out_shape=[
            jax.ShapeDtypeStruct((K, n), jnp.int32),
            jax.ShapeDtypeStruct((K, n), jnp.int32),
        ],
    )(bm_t)
    sel_t = sel.T
    rowids = rows.T.reshape(n * K)

    table = sims.reshape(nb * n, BLOCK_V)
    gath = _sc_gather(table, rowids, n)

    nqt = min(n, 64)
    nt = n // (2 * nqt)
    fvals, fidx = pl.pallas_call(
        _final_kernel,
        grid=(2, nt),
        in_specs=[
            pl.BlockSpec((nqt * K, BLOCK_V), lambda h, t: (h * nt + t, 0)),
            pl.BlockSpec((nqt, K), lambda h, t: (h * nt + t, 0)),
        ],
        out_specs=[
            pl.BlockSpec((1, K, nqt), lambda h, t: (h * nt + t, 0, 0)),
            pl.BlockSpec((1, K, nqt), lambda h, t: (h * nt + t, 0, 0)),
        ],
        out_shape=[
            jax.ShapeDtypeStruct((2 * nt, K, nqt), jnp.float32),
            jax.ShapeDtypeStruct((2 * nt, K, nqt), jnp.int32),
        ],
        compiler_params=pltpu.CompilerParams(
            dimension_semantics=("parallel", "arbitrary")),
    )(gath, sel_t)

    fv = fvals.transpose(0, 2, 1).reshape(n, K)
    fi = fidx.transpose(0, 2, 1).reshape(n, K)
    topk_sim = fv.reshape(orig_shape + (K,))
    topk_idx = fi.reshape(orig_shape + (K,))
    token_ids = fi[:, 0].reshape(orig_shape)
    return (token_ids, topk_sim, topk_idx)
